# baseline (speedup 1.0000x reference)
import functools

import jax
import jax.numpy as jnp
from jax import lax
from jax.experimental import pallas as pl
from jax.experimental.pallas import tpu as pltpu
from jax.experimental.pallas import tpu_sc as plsc

_NC = 2
_NS = 16
_LANES = 16
_K = 128
_D = 128
_DH = _D // _NC


def _sc_propagate(n_pad: int, n_chunks: int):
  mesh = plsc.VectorSubcoreMesh(core_axis_name="c", subcore_axis_name="s")
  per_tile = n_pad // _NS
  n_full = per_tile // _K
  tail = per_tile - n_full * _K
  assert n_chunks % 2 == 0

  @functools.partial(
      pl.kernel,
      out_type=jax.ShapeDtypeStruct((_NC, n_pad, _D), jnp.float32),
      mesh=mesh,
      compiler_params=pltpu.CompilerParams(use_tc_tiling_on_sc=False),
      scratch_types=[
          pltpu.VMEM((_K,), jnp.int32),
          pltpu.VMEM((_K,), jnp.int32),
          pltpu.VMEM((_K,), jnp.int32),
          pltpu.VMEM((_K,), jnp.int32),
          pltpu.VMEM((_K,), jnp.float32),
          pltpu.VMEM((_K,), jnp.float32),
          pltpu.VMEM((_K, _D), jnp.float32),
          pltpu.VMEM((_K, _D), jnp.float32),
          pltpu.VMEM_SHARED((n_pad, _D), jnp.float32),
          pltpu.SemaphoreType.DMA,
          pltpu.SemaphoreType.DMA,
          pltpu.SemaphoreType.DMA,
          pltpu.SemaphoreType.DMA,
          pltpu.SemaphoreType.DMA,
          pltpu.SemaphoreType.DMA,
      ],
  )
  def body(emb_hbm, src_hbm, dst_hbm, ev_hbm, out_hbm,
           src_v0, src_v1, dst_v0, dst_v1, ev_v0, ev_v1,
           rows_v0, rows_v1, acc_sh,
           gsem0, gsem1, sesem0, sesem1, dsem0, dsem1):
    cid = lax.axis_index("c")
    sid = lax.axis_index("s")
    base = sid * per_tile
    etab = emb_hbm
    srcs = (src_v0, src_v1)
    dsts = (dst_v0, dst_v1)
    evs = (ev_v0, ev_v1)
    rows = (rows_v0, rows_v1)
    gsems = (gsem0, gsem1)
    sesems = (sesem0, sesem1)
    dsems = (dsem0, dsem1)

    def srcev_start(c, b):
      pltpu.async_copy(src_hbm.at[cid, sid, c], srcs[b], sesems[b])
      pltpu.async_copy(ev_hbm.at[cid, sid, c], evs[b], sesems[b])

    def srcev_wait(b):
      pltpu.make_async_copy(src_hbm.at[cid, sid, 0], srcs[b], sesems[b]).wait()
      pltpu.make_async_copy(ev_hbm.at[cid, sid, 0], evs[b], sesems[b]).wait()

    def dst_start(c, b):
      pltpu.async_copy(dst_hbm.at[cid, sid, c], dsts[b], dsems[b])

    def dst_wait(b):
      pltpu.make_async_copy(dst_hbm.at[cid, sid, 0], dsts[b], dsems[b]).wait()

    def gather_start(b):
      pltpu.async_copy(etab.at[srcs[b]], rows[b], gsems[b])

    def gather_wait(b):
      pltpu.make_async_copy(etab.at[srcs[b]], rows[b], gsems[b]).wait()

    def scale(b):
      rv, ev = rows[b], evs[b]

      @plsc.parallel_loop(0, _K, step=_LANES, unroll=2)
      def _(r):
        ev16 = ev[pl.ds(r, _LANES)]
        for j in range(_LANES):
          e = ev16[j]
          for f in range(_D // _LANES):
            sl = pl.ds(f * _LANES, _LANES)
            rv[r + j, sl] = rv[r + j, sl] * e

    zvec = jnp.zeros((_LANES,), jnp.float32)

    def zero_row(i, carry):
      for j in range(_D // _LANES):
        rows_v0[i, pl.ds(j * _LANES, _LANES)] = zvec
      return carry

    lax.fori_loop(0, _K, zero_row, 0)
    for r in range(n_full):
      pltpu.sync_copy(rows_v0, acc_sh.at[pl.ds(base + r * _K, _K)])
    if tail:
      pltpu.sync_copy(rows_v0.at[pl.ds(0, tail)],
                      acc_sh.at[pl.ds(base + n_full * _K, tail)])
    plsc.subcore_barrier()

    srcev_start(0, 0)
    dst_start(0, 0)
    srcev_start(1, 1)
    dst_start(1, 1)
    srcev_wait(0)
    gather_start(0)

    def half(c_next, b):
      ob = 1 - b
      srcev_wait(ob)
      gather_start(ob)
      gather_wait(b)
      srcev_start(c_next, b)
      scale(b)
      dst_wait(b)
      pltpu.sync_copy(rows[b], acc_sh.at[dsts[b]], add=True)
      dst_start(c_next, b)

    def pair(c2, carry):
      c = c2 * 2
      half(c + 2, 0)
      half(c + 3, 1)
      return carry

    lax.fori_loop(0, n_chunks // 2, pair, 0)
    gather_wait(0)
    srcev_wait(1)
    dst_wait(0)
    dst_wait(1)
    plsc.subcore_barrier()

    for r in range(n_full):
      rs = pl.ds(base + r * _K, _K)
      pltpu.sync_copy(acc_sh.at[rs], rows_v0)
      pltpu.sync_copy(rows_v0, out_hbm.at[cid, rs])
    if tail:
      rs = pl.ds(base + n_full * _K, tail)
      pltpu.sync_copy(acc_sh.at[rs], rows_v0.at[pl.ds(0, tail)])
      pltpu.sync_copy(rows_v0.at[pl.ds(0, tail)], out_hbm.at[cid, rs])

  return body


def _tc_layer(parts, emb_prev, sum_prev, w, b2d, last: bool):
  n = emb_prev.shape[0]
  blk = 1000
  grid = (n // blk,)

  def body(p_ref, e_ref, s_ref, w_ref, b_ref, out_ref, sum_ref):
    agg = p_ref[0] + p_ref[1]
    h = jnp.dot(agg, w_ref[...], preferred_element_type=jnp.float32,
                precision=lax.Precision.HIGHEST)
    h = h + b_ref[...]
    if not last:
      h = jnp.maximum(h, 0.0) + e_ref[...]
    out_ref[...] = h
    sum_ref[...] = s_ref[...] + h

  return pl.pallas_call(
      body,
      grid=grid,
      in_specs=[
          pl.BlockSpec((_NC, blk, _D), lambda i: (0, i, 0)),
          pl.BlockSpec((blk, _D), lambda i: (i, 0)),
          pl.BlockSpec((blk, _D), lambda i: (i, 0)),
          pl.BlockSpec((_D, _D), lambda i: (0, 0)),
          pl.BlockSpec((1, _D), lambda i: (0, 0)),
      ],
      out_specs=[
          pl.BlockSpec((blk, _D), lambda i: (i, 0)),
          pl.BlockSpec((blk, _D), lambda i: (i, 0)),
      ],
      out_shape=[
          jax.ShapeDtypeStruct((n, _D), jnp.float32),
          jax.ShapeDtypeStruct((n, _D), jnp.float32),
      ],
  )(parts, emb_prev, sum_prev, w, b2d)


def _tc_user_gather(users2d, sum_users):
  bu, nu = users2d.shape[0], sum_users.shape[0]

  def body(u_ref, s_ref, o_ref):
    cols = lax.broadcasted_iota(jnp.int32, (bu, nu), 1)
    oh = jnp.where(cols == u_ref[...], 1.0, 0.0).astype(jnp.float32)
    o_ref[...] = jnp.dot(oh, s_ref[...], preferred_element_type=jnp.float32,
                         precision=lax.Precision.HIGHEST)

  return pl.pallas_call(
      body,
      out_shape=jax.ShapeDtypeStruct((bu, _D), jnp.float32),
  )(users2d, sum_users)


def _tc_rating(ue, sum_items):
  bu = ue.shape[0]
  ni = sum_items.shape[0]
  blk = 1024
  grid = (ni // blk,)

  def body(u_ref, it_ref, o_ref):
    z = lax.dot_general(
        u_ref[...], it_ref[...],
        dimension_numbers=(((1,), (1,)), ((), ())),
        preferred_element_type=jnp.float32,
        precision=lax.Precision.HIGHEST,
    ) * (1.0 / 16.0)
    o_ref[...] = 1.0 / (1.0 + jnp.exp(-z))

  return pl.pallas_call(
      body,
      grid=grid,
      in_specs=[
          pl.BlockSpec((bu, _D), lambda i: (0, 0)),
          pl.BlockSpec((blk, _D), lambda i: (i, 0)),
      ],
      out_specs=pl.BlockSpec((bu, blk), lambda i: (0, i)),
      out_shape=jax.ShapeDtypeStruct((bu, ni), jnp.float32),
  )(ue, sum_items)


def kernel(users, edge_index, edge_values, user_emb, item_emb,
           W0, b0, W1, b1, W2, b2):
  num_users = user_emb.shape[0]
  n_nodes = num_users + item_emb.shape[0]
  e = edge_index.shape[1]

  per = _NC * _NS * _K
  n_chunks = -(-e // per)
  n_chunks += n_chunks % 2
  epad = n_chunks * per
  pad = epad - e
  src = jnp.concatenate([edge_index[0], jnp.zeros((pad,), jnp.int32)])
  dst = jnp.concatenate([edge_index[1], jnp.zeros((pad,), jnp.int32)])
  ev = jnp.concatenate([edge_values, jnp.zeros((pad,), jnp.float32)])
  zc_i = jnp.zeros((_NC, _NS, 2, _K), jnp.int32)
  srcs = jnp.concatenate([src.reshape(_NC, _NS, n_chunks, _K), zc_i], axis=2)
  dsts = jnp.concatenate([dst.reshape(_NC, _NS, n_chunks, _K), zc_i], axis=2)
  evc = jnp.concatenate(
      [ev.reshape(_NC, _NS, n_chunks, _K),
       jnp.zeros((_NC, _NS, 2, _K), jnp.float32)], axis=2)

  emb = jnp.concatenate([user_emb, item_emb], axis=0)
  sum_ = emb
  n_pad = -(-n_nodes // (_NS * 8)) * (_NS * 8)
  sc_prop = _sc_propagate(n_pad, n_chunks)
  for w, b, last in ((W0, b0, False), (W1, b1, False), (W2, b2, True)):
    parts = sc_prop(emb, srcs, dsts, evc)
    emb, sum_ = _tc_layer(parts[:, :n_nodes, :], emb, sum_,
                          w, b.reshape(1, _D), last)

  ue = _tc_user_gather(users.reshape(-1, 1), sum_[:num_users])
  n_items = n_nodes - num_users
  ni_pad = -(-n_items // 1024) * 1024
  items = jnp.concatenate(
      [sum_[num_users:],
       jnp.zeros((ni_pad - n_items, _D), jnp.float32)], axis=0)
  return _tc_rating(ue, items)[:, :n_items]


# speedup vs baseline: 1.1964x; 1.1964x over previous
import functools

import jax
import jax.numpy as jnp
from jax import lax
from jax.experimental import pallas as pl
from jax.experimental.pallas import tpu as pltpu
from jax.experimental.pallas import tpu_sc as plsc

_NC = 2
_NS = 16
_LANES = 16
_K = 128
_D = 128
_DH = _D // _NC


def _sc_propagate(n_pad: int, n_chunks: int):
  mesh = plsc.VectorSubcoreMesh(core_axis_name="c", subcore_axis_name="s")
  per_tile = n_pad // _NS
  n_full = per_tile // _K
  tail = per_tile - n_full * _K
  assert n_chunks % 2 == 0

  @functools.partial(
      pl.kernel,
      out_type=jax.ShapeDtypeStruct((_NC, n_pad, _DH), jnp.float32),
      mesh=mesh,
      compiler_params=pltpu.CompilerParams(use_tc_tiling_on_sc=False),
      scratch_types=[
          pltpu.VMEM((_K,), jnp.int32),
          pltpu.VMEM((_K,), jnp.int32),
          pltpu.VMEM((_K,), jnp.int32),
          pltpu.VMEM((_K,), jnp.int32),
          pltpu.VMEM((_K,), jnp.float32),
          pltpu.VMEM((_K,), jnp.float32),
          pltpu.VMEM((_K, _DH), jnp.float32),
          pltpu.VMEM((_K, _DH), jnp.float32),
          pltpu.VMEM_SHARED((n_pad, _DH), jnp.float32),
          pltpu.SemaphoreType.DMA,
          pltpu.SemaphoreType.DMA,
          pltpu.SemaphoreType.DMA,
          pltpu.SemaphoreType.DMA,
          pltpu.SemaphoreType.DMA,
          pltpu.SemaphoreType.DMA,
      ],
  )
  def body(emb_hbm, src_hbm, dst_hbm, ev_hbm, out_hbm,
           src_v0, src_v1, dst_v0, dst_v1, ev_v0, ev_v1,
           rows_v0, rows_v1, acc_sh,
           gsem0, gsem1, sesem0, sesem1, dsem0, dsem1):
    cid = lax.axis_index("c")
    sid = lax.axis_index("s")
    base = sid * per_tile
    etab = emb_hbm.at[cid]
    srcs = (src_v0, src_v1)
    dsts = (dst_v0, dst_v1)
    evs = (ev_v0, ev_v1)
    rows = (rows_v0, rows_v1)
    gsems = (gsem0, gsem1)
    sesems = (sesem0, sesem1)
    dsems = (dsem0, dsem1)

    def srcev_start(c, b):
      pltpu.async_copy(src_hbm.at[sid, c], srcs[b], sesems[b])
      pltpu.async_copy(ev_hbm.at[sid, c], evs[b], sesems[b])

    def srcev_wait(b):
      pltpu.make_async_copy(src_hbm.at[sid, 0], srcs[b], sesems[b]).wait()
      pltpu.make_async_copy(ev_hbm.at[sid, 0], evs[b], sesems[b]).wait()

    def dst_start(c, b):
      pltpu.async_copy(dst_hbm.at[sid, c], dsts[b], dsems[b])

    def dst_wait(b):
      pltpu.make_async_copy(dst_hbm.at[sid, 0], dsts[b], dsems[b]).wait()

    def gather_start(b):
      pltpu.async_copy(etab.at[srcs[b]], rows[b], gsems[b])

    def gather_wait(b):
      pltpu.make_async_copy(etab.at[srcs[b]], rows[b], gsems[b]).wait()

    def scale(b):
      rv, ev = rows[b], evs[b]

      def scale16(i, carry2):
        r = i * _LANES
        ev16 = ev[pl.ds(r, _LANES)]
        for j in range(_LANES):
          e = ev16[j]
          for f in range(_DH // _LANES):
            sl = pl.ds(f * _LANES, _LANES)
            rv[r + j, sl] = rv[r + j, sl] * e
        return carry2

      lax.fori_loop(0, _K // _LANES, scale16, 0)

    zvec = jnp.zeros((_LANES,), jnp.float32)

    def zero_row(i, carry):
      for j in range(_DH // _LANES):
        rows_v0[i, pl.ds(j * _LANES, _LANES)] = zvec
      return carry

    lax.fori_loop(0, _K, zero_row, 0)
    for r in range(n_full):
      pltpu.sync_copy(rows_v0, acc_sh.at[pl.ds(base + r * _K, _K)])
    if tail:
      pltpu.sync_copy(rows_v0.at[pl.ds(0, tail)],
                      acc_sh.at[pl.ds(base + n_full * _K, tail)])
    plsc.subcore_barrier()

    srcev_start(0, 0)
    dst_start(0, 0)
    srcev_start(1, 1)
    dst_start(1, 1)
    srcev_wait(0)
    gather_start(0)

    def half(c_next, b):
      ob = 1 - b
      srcev_wait(ob)
      gather_start(ob)
      gather_wait(b)
      srcev_start(c_next, b)
      scale(b)
      dst_wait(b)
      pltpu.sync_copy(rows[b], acc_sh.at[dsts[b]], add=True)
      dst_start(c_next, b)

    def pair(c2, carry):
      c = c2 * 2
      half(c + 2, 0)
      half(c + 3, 1)
      return carry

    lax.fori_loop(0, n_chunks // 2, pair, 0)
    gather_wait(0)
    srcev_wait(1)
    dst_wait(0)
    dst_wait(1)
    plsc.subcore_barrier()

    for r in range(n_full):
      rs = pl.ds(base + r * _K, _K)
      pltpu.sync_copy(acc_sh.at[rs], rows_v0)
      pltpu.sync_copy(rows_v0, out_hbm.at[cid, rs])
    if tail:
      rs = pl.ds(base + n_full * _K, tail)
      pltpu.sync_copy(acc_sh.at[rs], rows_v0.at[pl.ds(0, tail)])
      pltpu.sync_copy(rows_v0.at[pl.ds(0, tail)], out_hbm.at[cid, rs])

  return body


def _tc_layer(parts, emb_prev, sum_prev, w, b2d, last: bool):
  n = emb_prev.shape[0]
  blk = 1000
  grid = (n // blk,)

  def body(p_ref, e_ref, s_ref, w_ref, b_ref, out_ref, sum_ref):
    agg = jnp.concatenate([p_ref[0], p_ref[1]], axis=1)
    h = jnp.dot(agg, w_ref[...], preferred_element_type=jnp.float32,
                precision=lax.Precision.HIGHEST)
    h = h + b_ref[...]
    if not last:
      h = jnp.maximum(h, 0.0) + e_ref[...]
    out_ref[...] = h
    sum_ref[...] = s_ref[...] + h

  return pl.pallas_call(
      body,
      grid=grid,
      in_specs=[
          pl.BlockSpec((_NC, blk, _DH), lambda i: (0, i, 0)),
          pl.BlockSpec((blk, _D), lambda i: (i, 0)),
          pl.BlockSpec((blk, _D), lambda i: (i, 0)),
          pl.BlockSpec((_D, _D), lambda i: (0, 0)),
          pl.BlockSpec((1, _D), lambda i: (0, 0)),
      ],
      out_specs=[
          pl.BlockSpec((blk, _D), lambda i: (i, 0)),
          pl.BlockSpec((blk, _D), lambda i: (i, 0)),
      ],
      out_shape=[
          jax.ShapeDtypeStruct((n, _D), jnp.float32),
          jax.ShapeDtypeStruct((n, _D), jnp.float32),
      ],
  )(parts, emb_prev, sum_prev, w, b2d)


def _tc_user_gather(users2d, sum_users):
  bu, nu = users2d.shape[0], sum_users.shape[0]

  def body(u_ref, s_ref, o_ref):
    cols = lax.broadcasted_iota(jnp.int32, (bu, nu), 1)
    oh = jnp.where(cols == u_ref[...], 1.0, 0.0).astype(jnp.float32)
    o_ref[...] = jnp.dot(oh, s_ref[...], preferred_element_type=jnp.float32,
                         precision=lax.Precision.HIGHEST)

  return pl.pallas_call(
      body,
      out_shape=jax.ShapeDtypeStruct((bu, _D), jnp.float32),
  )(users2d, sum_users)


def _tc_rating(ue, sum_items):
  bu = ue.shape[0]
  ni = sum_items.shape[0]
  blk = 1024
  grid = (ni // blk,)

  def body(u_ref, it_ref, o_ref):
    z = lax.dot_general(
        u_ref[...], it_ref[...],
        dimension_numbers=(((1,), (1,)), ((), ())),
        preferred_element_type=jnp.float32,
        precision=lax.Precision.HIGHEST,
    ) * (1.0 / 16.0)
    o_ref[...] = 1.0 / (1.0 + jnp.exp(-z))

  return pl.pallas_call(
      body,
      grid=grid,
      in_specs=[
          pl.BlockSpec((bu, _D), lambda i: (0, 0)),
          pl.BlockSpec((blk, _D), lambda i: (i, 0)),
      ],
      out_specs=pl.BlockSpec((bu, blk), lambda i: (0, i)),
      out_shape=jax.ShapeDtypeStruct((bu, ni), jnp.float32),
  )(ue, sum_items)


def kernel(users, edge_index, edge_values, user_emb, item_emb,
           W0, b0, W1, b1, W2, b2):
  num_users = user_emb.shape[0]
  n_nodes = num_users + item_emb.shape[0]
  e = edge_index.shape[1]

  per = _NS * _K
  n_chunks = -(-e // per)
  n_chunks += n_chunks % 2
  epad = n_chunks * per
  pad = epad - e
  src = jnp.concatenate([edge_index[0], jnp.zeros((pad,), jnp.int32)])
  dst = jnp.concatenate([edge_index[1], jnp.zeros((pad,), jnp.int32)])
  ev = jnp.concatenate([edge_values, jnp.zeros((pad,), jnp.float32)])
  zc_i = jnp.zeros((_NS, 2, _K), jnp.int32)
  srcs = jnp.concatenate([src.reshape(_NS, n_chunks, _K), zc_i], axis=1)
  dsts = jnp.concatenate([dst.reshape(_NS, n_chunks, _K), zc_i], axis=1)
  evc = jnp.concatenate(
      [ev.reshape(_NS, n_chunks, _K), jnp.zeros((_NS, 2, _K), jnp.float32)],
      axis=1)

  emb = jnp.concatenate([user_emb, item_emb], axis=0)
  sum_ = emb
  n_pad = -(-n_nodes // (_NS * 8)) * (_NS * 8)
  sc_prop = _sc_propagate(n_pad, n_chunks)
  for w, b, last in ((W0, b0, False), (W1, b1, False), (W2, b2, True)):
    emb2 = emb.reshape(n_nodes, _NC, _DH).transpose(1, 0, 2)
    parts = sc_prop(emb2, srcs, dsts, evc)
    emb, sum_ = _tc_layer(parts[:, :n_nodes, :], emb, sum_,
                          w, b.reshape(1, _D), last)

  ue = _tc_user_gather(users.reshape(-1, 1), sum_[:num_users])
  n_items = n_nodes - num_users
  ni_pad = -(-n_items // 1024) * 1024
  items = jnp.concatenate(
      [sum_[num_users:],
       jnp.zeros((ni_pad - n_items, _D), jnp.float32)], axis=0)
  return _tc_rating(ue, items)[:, :n_items]


# speedup vs baseline: 1.8842x; 1.5749x over previous
import functools

import jax
import jax.numpy as jnp
from jax import lax
from jax.experimental import pallas as pl
from jax.experimental.pallas import tpu as pltpu
from jax.experimental.pallas import tpu_sc as plsc

_NC = 2
_NS = 16
_LANES = 16
_K = 128
_D = 128
_DH = _D // _NC


def _sc_propagate(n_pad: int, n_chunks: int):
  mesh = plsc.VectorSubcoreMesh(core_axis_name="c", subcore_axis_name="s")
  per_tile = n_pad // _NS
  n_full = per_tile // _K
  tail = per_tile - n_full * _K
  assert n_chunks % 2 == 0

  @functools.partial(
      pl.kernel,
      out_type=jax.ShapeDtypeStruct((_NC, n_pad, _DH), jnp.float32),
      mesh=mesh,
      compiler_params=pltpu.CompilerParams(use_tc_tiling_on_sc=False),
      scratch_types=[
          pltpu.VMEM((_K,), jnp.int32),
          pltpu.VMEM((_K,), jnp.int32),
          pltpu.VMEM((_K,), jnp.int32),
          pltpu.VMEM((_K,), jnp.int32),
          pltpu.VMEM((_K,), jnp.float32),
          pltpu.VMEM((_K,), jnp.float32),
          pltpu.VMEM((_K, _DH), jnp.float32),
          pltpu.VMEM((_K, _DH), jnp.float32),
          pltpu.VMEM_SHARED((n_pad, _DH), jnp.float32),
          pltpu.SemaphoreType.DMA,
          pltpu.SemaphoreType.DMA,
          pltpu.SemaphoreType.DMA,
          pltpu.SemaphoreType.DMA,
          pltpu.SemaphoreType.DMA,
          pltpu.SemaphoreType.DMA,
      ],
  )
  def body(emb_hbm, src_hbm, dst_hbm, ev_hbm, out_hbm,
           src_v0, src_v1, dst_v0, dst_v1, ev_v0, ev_v1,
           rows_v0, rows_v1, acc_sh,
           gsem0, gsem1, sesem0, sesem1, dsem0, dsem1):
    cid = lax.axis_index("c")
    sid = lax.axis_index("s")
    base = sid * per_tile
    etab = emb_hbm.at[cid]
    srcs = (src_v0, src_v1)
    dsts = (dst_v0, dst_v1)
    evs = (ev_v0, ev_v1)
    rows = (rows_v0, rows_v1)
    gsems = (gsem0, gsem1)
    sesems = (sesem0, sesem1)
    dsems = (dsem0, dsem1)

    def srcev_start(c, b):
      pltpu.async_copy(src_hbm.at[sid, c], srcs[b], sesems[b])
      pltpu.async_copy(ev_hbm.at[sid, c], evs[b], sesems[b])

    def srcev_wait(b):
      pltpu.make_async_copy(src_hbm.at[sid, 0], srcs[b], sesems[b]).wait()
      pltpu.make_async_copy(ev_hbm.at[sid, 0], evs[b], sesems[b]).wait()

    def dst_start(c, b):
      pltpu.async_copy(dst_hbm.at[sid, c], dsts[b], dsems[b])

    def dst_wait(b):
      pltpu.make_async_copy(dst_hbm.at[sid, 0], dsts[b], dsems[b]).wait()

    def gather_start(b):
      pltpu.async_copy(etab.at[srcs[b]], rows[b], gsems[b])

    def gather_wait(b):
      pltpu.make_async_copy(etab.at[srcs[b]], rows[b], gsems[b]).wait()

    def scale(b):
      rv, ev = rows[b], evs[b]

      @plsc.parallel_loop(0, _K, step=_LANES, unroll=2)
      def _(r):
        ev16 = ev[pl.ds(r, _LANES)]
        for j in range(_LANES):
          e = ev16[j]
          for f in range(_DH // _LANES):
            sl = pl.ds(f * _LANES, _LANES)
            rv[r + j, sl] = rv[r + j, sl] * e

    zvec = jnp.zeros((_LANES,), jnp.float32)

    def zero_row(i, carry):
      for j in range(_DH // _LANES):
        rows_v0[i, pl.ds(j * _LANES, _LANES)] = zvec
      return carry

    lax.fori_loop(0, _K, zero_row, 0)
    for r in range(n_full):
      pltpu.sync_copy(rows_v0, acc_sh.at[pl.ds(base + r * _K, _K)])
    if tail:
      pltpu.sync_copy(rows_v0.at[pl.ds(0, tail)],
                      acc_sh.at[pl.ds(base + n_full * _K, tail)])
    plsc.subcore_barrier()

    srcev_start(0, 0)
    dst_start(0, 0)
    srcev_start(1, 1)
    dst_start(1, 1)
    srcev_wait(0)
    gather_start(0)

    def half(c_next, b):
      ob = 1 - b
      srcev_wait(ob)
      gather_start(ob)
      gather_wait(b)
      srcev_start(c_next, b)
      scale(b)
      dst_wait(b)
      pltpu.sync_copy(rows[b], acc_sh.at[dsts[b]], add=True)
      dst_start(c_next, b)

    def pair(c2, carry):
      c = c2 * 2
      half(c + 2, 0)
      half(c + 3, 1)
      return carry

    lax.fori_loop(0, n_chunks // 2, pair, 0)
    gather_wait(0)
    srcev_wait(1)
    dst_wait(0)
    dst_wait(1)
    plsc.subcore_barrier()

    for r in range(n_full):
      rs = pl.ds(base + r * _K, _K)
      pltpu.sync_copy(acc_sh.at[rs], rows_v0)
      pltpu.sync_copy(rows_v0, out_hbm.at[cid, rs])
    if tail:
      rs = pl.ds(base + n_full * _K, tail)
      pltpu.sync_copy(acc_sh.at[rs], rows_v0.at[pl.ds(0, tail)])
      pltpu.sync_copy(rows_v0.at[pl.ds(0, tail)], out_hbm.at[cid, rs])

  return body


def _tc_layer(parts, emb_prev, sum_prev, w, b2d, last: bool):
  n = emb_prev.shape[0]
  blk = 1000
  grid = (n // blk,)

  def body(p_ref, e_ref, s_ref, w_ref, b_ref, out_ref, sum_ref):
    agg = jnp.concatenate([p_ref[0], p_ref[1]], axis=1)
    h = jnp.dot(agg, w_ref[...], preferred_element_type=jnp.float32,
                precision=lax.Precision.HIGHEST)
    h = h + b_ref[...]
    if not last:
      h = jnp.maximum(h, 0.0) + e_ref[...]
    out_ref[...] = h
    sum_ref[...] = s_ref[...] + h

  return pl.pallas_call(
      body,
      grid=grid,
      in_specs=[
          pl.BlockSpec((_NC, blk, _DH), lambda i: (0, i, 0)),
          pl.BlockSpec((blk, _D), lambda i: (i, 0)),
          pl.BlockSpec((blk, _D), lambda i: (i, 0)),
          pl.BlockSpec((_D, _D), lambda i: (0, 0)),
          pl.BlockSpec((1, _D), lambda i: (0, 0)),
      ],
      out_specs=[
          pl.BlockSpec((blk, _D), lambda i: (i, 0)),
          pl.BlockSpec((blk, _D), lambda i: (i, 0)),
      ],
      out_shape=[
          jax.ShapeDtypeStruct((n, _D), jnp.float32),
          jax.ShapeDtypeStruct((n, _D), jnp.float32),
      ],
  )(parts, emb_prev, sum_prev, w, b2d)


def _tc_user_gather(users2d, sum_users):
  bu, nu = users2d.shape[0], sum_users.shape[0]

  def body(u_ref, s_ref, o_ref):
    cols = lax.broadcasted_iota(jnp.int32, (bu, nu), 1)
    oh = jnp.where(cols == u_ref[...], 1.0, 0.0).astype(jnp.float32)
    o_ref[...] = jnp.dot(oh, s_ref[...], preferred_element_type=jnp.float32,
                         precision=lax.Precision.HIGHEST)

  return pl.pallas_call(
      body,
      out_shape=jax.ShapeDtypeStruct((bu, _D), jnp.float32),
  )(users2d, sum_users)


def _tc_rating(ue, sum_items):
  bu = ue.shape[0]
  ni = sum_items.shape[0]
  blk = 1024
  grid = (ni // blk,)

  def body(u_ref, it_ref, o_ref):
    z = lax.dot_general(
        u_ref[...], it_ref[...],
        dimension_numbers=(((1,), (1,)), ((), ())),
        preferred_element_type=jnp.float32,
        precision=lax.Precision.HIGHEST,
    ) * (1.0 / 16.0)
    o_ref[...] = 1.0 / (1.0 + jnp.exp(-z))

  return pl.pallas_call(
      body,
      grid=grid,
      in_specs=[
          pl.BlockSpec((bu, _D), lambda i: (0, 0)),
          pl.BlockSpec((blk, _D), lambda i: (i, 0)),
      ],
      out_specs=pl.BlockSpec((bu, blk), lambda i: (0, i)),
      out_shape=jax.ShapeDtypeStruct((bu, ni), jnp.float32),
  )(ue, sum_items)


def kernel(users, edge_index, edge_values, user_emb, item_emb,
           W0, b0, W1, b1, W2, b2):
  num_users = user_emb.shape[0]
  n_nodes = num_users + item_emb.shape[0]
  e = edge_index.shape[1]

  per = _NS * _K
  n_chunks = -(-e // per)
  n_chunks += n_chunks % 2
  epad = n_chunks * per
  pad = epad - e
  src = jnp.concatenate([edge_index[0], jnp.zeros((pad,), jnp.int32)])
  dst = jnp.concatenate([edge_index[1], jnp.zeros((pad,), jnp.int32)])
  ev = jnp.concatenate([edge_values, jnp.zeros((pad,), jnp.float32)])
  zc_i = jnp.zeros((_NS, 2, _K), jnp.int32)
  srcs = jnp.concatenate([src.reshape(_NS, n_chunks, _K), zc_i], axis=1)
  dsts = jnp.concatenate([dst.reshape(_NS, n_chunks, _K), zc_i], axis=1)
  evc = jnp.concatenate(
      [ev.reshape(_NS, n_chunks, _K), jnp.zeros((_NS, 2, _K), jnp.float32)],
      axis=1)

  emb = jnp.concatenate([user_emb, item_emb], axis=0)
  sum_ = emb
  n_pad = -(-n_nodes // (_NS * 8)) * (_NS * 8)
  sc_prop = _sc_propagate(n_pad, n_chunks)
  for w, b, last in ((W0, b0, False), (W1, b1, False), (W2, b2, True)):
    emb2 = emb.reshape(n_nodes, _NC, _DH).transpose(1, 0, 2)
    parts = sc_prop(emb2, srcs, dsts, evc)
    emb, sum_ = _tc_layer(parts[:, :n_nodes, :], emb, sum_,
                          w, b.reshape(1, _D), last)

  ue = _tc_user_gather(users.reshape(-1, 1), sum_[:num_users])
  n_items = n_nodes - num_users
  ni_pad = -(-n_items // 1024) * 1024
  items = jnp.concatenate(
      [sum_[num_users:],
       jnp.zeros((ni_pad - n_items, _D), jnp.float32)], axis=0)
  return _tc_rating(ue, items)[:, :n_items]


# speedup vs baseline: 2.0927x; 1.1107x over previous
import functools

import jax
import jax.numpy as jnp
from jax import lax
from jax.experimental import pallas as pl
from jax.experimental.pallas import tpu as pltpu
from jax.experimental.pallas import tpu_sc as plsc

_NC = 2
_NS = 16
_LANES = 16
_K = 128
_D = 128
_DH = _D // _NC


def _sc_propagate(n_pad: int, n_chunks: int):
  mesh = plsc.VectorSubcoreMesh(core_axis_name="c", subcore_axis_name="s")
  per_tile = n_pad // _NS
  n_full = per_tile // _K
  tail = per_tile - n_full * _K
  assert n_chunks % 2 == 0

  @functools.partial(
      pl.kernel,
      out_type=jax.ShapeDtypeStruct((_NC, n_pad, _DH), jnp.float32),
      mesh=mesh,
      compiler_params=pltpu.CompilerParams(use_tc_tiling_on_sc=False),
      scratch_types=[
          pltpu.VMEM((_K,), jnp.int32),
          pltpu.VMEM((_K,), jnp.int32),
          pltpu.VMEM((_K,), jnp.int32),
          pltpu.VMEM((_K,), jnp.int32),
          pltpu.VMEM((_K,), jnp.float32),
          pltpu.VMEM((_K,), jnp.float32),
          pltpu.VMEM((_K, _DH), jnp.float32),
          pltpu.VMEM((_K, _DH), jnp.float32),
          pltpu.VMEM_SHARED((n_pad, _DH), jnp.float32),
          pltpu.SemaphoreType.DMA,
          pltpu.SemaphoreType.DMA,
          pltpu.SemaphoreType.DMA,
          pltpu.SemaphoreType.DMA,
          pltpu.SemaphoreType.DMA,
          pltpu.SemaphoreType.DMA,
          pltpu.SemaphoreType.DMA,
          pltpu.SemaphoreType.DMA,
      ],
  )
  def body(emb_hbm, src_hbm, dst_hbm, ev_hbm, out_hbm,
           src_v0, src_v1, dst_v0, dst_v1, ev_v0, ev_v1,
           rows_v0, rows_v1, acc_sh,
           gsem0, gsem1, sesem0, sesem1, dsem0, dsem1, ssem0, ssem1):
    cid = lax.axis_index("c")
    sid = lax.axis_index("s")
    base = sid * per_tile
    etab = emb_hbm.at[cid]
    srcs = (src_v0, src_v1)
    dsts = (dst_v0, dst_v1)
    evs = (ev_v0, ev_v1)
    rows = (rows_v0, rows_v1)
    gsems = (gsem0, gsem1)
    sesems = (sesem0, sesem1)
    dsems = (dsem0, dsem1)
    ssems = (ssem0, ssem1)

    def srcev_start(c, b):
      pltpu.async_copy(src_hbm.at[sid, c], srcs[b], sesems[b])
      pltpu.async_copy(ev_hbm.at[sid, c], evs[b], sesems[b])

    def srcev_wait(b):
      pltpu.make_async_copy(src_hbm.at[sid, 0], srcs[b], sesems[b]).wait()
      pltpu.make_async_copy(ev_hbm.at[sid, 0], evs[b], sesems[b]).wait()

    def dst_start(c, b):
      pltpu.async_copy(dst_hbm.at[sid, c], dsts[b], dsems[b])

    def dst_wait(b):
      pltpu.make_async_copy(dst_hbm.at[sid, 0], dsts[b], dsems[b]).wait()

    def gather_start(b):
      pltpu.async_copy(etab.at[srcs[b]], rows[b], gsems[b])

    def gather_wait(b):
      pltpu.make_async_copy(etab.at[srcs[b]], rows[b], gsems[b]).wait()

    def scat_start(b):
      pltpu.async_copy(rows[b], acc_sh.at[dsts[b]], ssems[b], add=True)

    def scat_wait(b):
      pltpu.make_async_copy(rows[b], acc_sh.at[dsts[b]], ssems[b]).wait()

    def scale(b):
      rv, ev = rows[b], evs[b]

      @plsc.parallel_loop(0, _K, step=_LANES, unroll=2)
      def _(r):
        ev16 = ev[pl.ds(r, _LANES)]
        for j in range(_LANES):
          e = ev16[j]
          for f in range(_DH // _LANES):
            sl = pl.ds(f * _LANES, _LANES)
            rv[r + j, sl] = rv[r + j, sl] * e

    zvec = jnp.zeros((_LANES,), jnp.float32)

    def zero_row(i, carry):
      for j in range(_DH // _LANES):
        rows_v0[i, pl.ds(j * _LANES, _LANES)] = zvec
      return carry

    lax.fori_loop(0, _K, zero_row, 0)
    for r in range(n_full):
      pltpu.sync_copy(rows_v0, acc_sh.at[pl.ds(base + r * _K, _K)])
    if tail:
      pltpu.sync_copy(rows_v0.at[pl.ds(0, tail)],
                      acc_sh.at[pl.ds(base + n_full * _K, tail)])
    plsc.subcore_barrier()

    srcev_start(0, 0)
    dst_start(0, 0)
    srcev_start(1, 1)
    dst_start(1, 1)
    srcev_wait(0)
    gather_start(0)

    srcev_wait(1)
    gather_start(1)
    gather_wait(0)
    srcev_start(2, 0)
    scale(0)
    dst_wait(0)
    scat_start(0)

    def half(c_next, b):
      ob = 1 - b
      srcev_wait(ob)
      scat_wait(ob)
      gather_start(ob)
      dst_start(c_next - 1, ob)
      gather_wait(b)
      srcev_start(c_next, b)
      scale(b)
      dst_wait(b)
      scat_start(b)

    def pair(i, carry):
      c = 2 * i + 1
      half(c + 2, 1)
      half(c + 3, 0)
      return carry

    lax.fori_loop(0, (n_chunks - 2) // 2, pair, 0)

    srcev_wait(0)
    scat_wait(0)
    gather_wait(1)
    scale(1)
    dst_wait(1)
    scat_start(1)
    scat_wait(1)
    plsc.subcore_barrier()

    for r in range(n_full):
      rs = pl.ds(base + r * _K, _K)
      pltpu.sync_copy(acc_sh.at[rs], rows_v0)
      pltpu.sync_copy(rows_v0, out_hbm.at[cid, rs])
    if tail:
      rs = pl.ds(base + n_full * _K, tail)
      pltpu.sync_copy(acc_sh.at[rs], rows_v0.at[pl.ds(0, tail)])
      pltpu.sync_copy(rows_v0.at[pl.ds(0, tail)], out_hbm.at[cid, rs])

  return body


def _tc_layer(parts, emb_prev, sum_prev, w, b2d, last: bool):
  n = emb_prev.shape[0]
  blk = 1000
  grid = (n // blk,)

  def body(p_ref, e_ref, s_ref, w_ref, b_ref, out_ref, sum_ref):
    agg = jnp.concatenate([p_ref[0], p_ref[1]], axis=1)
    h = jnp.dot(agg, w_ref[...], preferred_element_type=jnp.float32,
                precision=lax.Precision.HIGHEST)
    h = h + b_ref[...]
    if not last:
      h = jnp.maximum(h, 0.0) + e_ref[...]
    out_ref[...] = h
    sum_ref[...] = s_ref[...] + h

  return pl.pallas_call(
      body,
      grid=grid,
      in_specs=[
          pl.BlockSpec((_NC, blk, _DH), lambda i: (0, i, 0)),
          pl.BlockSpec((blk, _D), lambda i: (i, 0)),
          pl.BlockSpec((blk, _D), lambda i: (i, 0)),
          pl.BlockSpec((_D, _D), lambda i: (0, 0)),
          pl.BlockSpec((1, _D), lambda i: (0, 0)),
      ],
      out_specs=[
          pl.BlockSpec((blk, _D), lambda i: (i, 0)),
          pl.BlockSpec((blk, _D), lambda i: (i, 0)),
      ],
      out_shape=[
          jax.ShapeDtypeStruct((n, _D), jnp.float32),
          jax.ShapeDtypeStruct((n, _D), jnp.float32),
      ],
  )(parts, emb_prev, sum_prev, w, b2d)


def _tc_user_gather(users2d, sum_users):
  bu, nu = users2d.shape[0], sum_users.shape[0]

  def body(u_ref, s_ref, o_ref):
    cols = lax.broadcasted_iota(jnp.int32, (bu, nu), 1)
    oh = jnp.where(cols == u_ref[...], 1.0, 0.0).astype(jnp.float32)
    o_ref[...] = jnp.dot(oh, s_ref[...], preferred_element_type=jnp.float32,
                         precision=lax.Precision.HIGHEST)

  return pl.pallas_call(
      body,
      out_shape=jax.ShapeDtypeStruct((bu, _D), jnp.float32),
  )(users2d, sum_users)


def _tc_rating(ue, sum_items):
  bu = ue.shape[0]
  ni = sum_items.shape[0]
  blk = 1024
  grid = (ni // blk,)

  def body(u_ref, it_ref, o_ref):
    z = lax.dot_general(
        u_ref[...], it_ref[...],
        dimension_numbers=(((1,), (1,)), ((), ())),
        preferred_element_type=jnp.float32,
        precision=lax.Precision.HIGHEST,
    ) * (1.0 / 16.0)
    o_ref[...] = 1.0 / (1.0 + jnp.exp(-z))

  return pl.pallas_call(
      body,
      grid=grid,
      in_specs=[
          pl.BlockSpec((bu, _D), lambda i: (0, 0)),
          pl.BlockSpec((blk, _D), lambda i: (i, 0)),
      ],
      out_specs=pl.BlockSpec((bu, blk), lambda i: (0, i)),
      out_shape=jax.ShapeDtypeStruct((bu, ni), jnp.float32),
  )(ue, sum_items)


def kernel(users, edge_index, edge_values, user_emb, item_emb,
           W0, b0, W1, b1, W2, b2):
  num_users = user_emb.shape[0]
  n_nodes = num_users + item_emb.shape[0]
  e = edge_index.shape[1]

  per = _NS * _K
  n_chunks = -(-e // per)
  n_chunks += n_chunks % 2
  epad = n_chunks * per
  pad = epad - e
  src = jnp.concatenate([edge_index[0], jnp.zeros((pad,), jnp.int32)])
  dst = jnp.concatenate([edge_index[1], jnp.zeros((pad,), jnp.int32)])
  ev = jnp.concatenate([edge_values, jnp.zeros((pad,), jnp.float32)])
  zc_i = jnp.zeros((_NS, 2, _K), jnp.int32)
  srcs = jnp.concatenate([src.reshape(_NS, n_chunks, _K), zc_i], axis=1)
  dsts = jnp.concatenate([dst.reshape(_NS, n_chunks, _K), zc_i], axis=1)
  evc = jnp.concatenate(
      [ev.reshape(_NS, n_chunks, _K), jnp.zeros((_NS, 2, _K), jnp.float32)],
      axis=1)

  emb = jnp.concatenate([user_emb, item_emb], axis=0)
  sum_ = emb
  n_pad = -(-n_nodes // (_NS * 8)) * (_NS * 8)
  sc_prop = _sc_propagate(n_pad, n_chunks)
  for w, b, last in ((W0, b0, False), (W1, b1, False), (W2, b2, True)):
    emb2 = emb.reshape(n_nodes, _NC, _DH).transpose(1, 0, 2)
    parts = sc_prop(emb2, srcs, dsts, evc)
    emb, sum_ = _tc_layer(parts[:, :n_nodes, :], emb, sum_,
                          w, b.reshape(1, _D), last)

  ue = _tc_user_gather(users.reshape(-1, 1), sum_[:num_users])
  n_items = n_nodes - num_users
  ni_pad = -(-n_items // 1024) * 1024
  items = jnp.concatenate(
      [sum_[num_users:],
       jnp.zeros((ni_pad - n_items, _D), jnp.float32)], axis=0)
  return _tc_rating(ue, items)[:, :n_items]
